# Initial kernel scaffold; baseline (speedup 1.0000x reference)
import jax
import jax.numpy as jnp
from jax.experimental import pallas as pl


def kernel(x, edge_index, edge_attr, W, b):
    raise NotImplementedError("write your pallas kernel here")


# speedup vs baseline: 8.3271x; 8.3271x over previous
import dataclasses
import functools

import jax
import jax.numpy as jnp
from jax import lax
from jax.experimental import pallas as pl
from jax.experimental.pallas import tpu as pltpu
from jax.experimental.pallas import tpu_sc as plsc

N = 10000
D = 128
NC = 2
NS = 16
NW = NC * NS
LANES = 16
N_PAD = 10240
ROWS_PER_TILE = N_PAD // NS
CHUNK = 128


def _ew_body(attr_ref, out_ref):
    v = attr_ref[...]
    out_ref[...] = (jnp.sum(v, axis=1) * 0.25)[None, None, :]


def _edge_weight(edge_attr):
    e, four = edge_attr.shape
    nblk = 40
    r = e // nblk
    out = pl.pallas_call(
        _ew_body,
        grid=(nblk,),
        in_specs=[pl.BlockSpec((r, four), lambda i: (i, 0))],
        out_specs=pl.BlockSpec((1, 1, r), lambda i: (i, 0, 0)),
        out_shape=jax.ShapeDtypeStruct((nblk, 1, r), jnp.float32),
    )(edge_attr)
    return out.reshape(e)


def _dis_body(degp_ref, out_ref):
    deg = degp_ref[0] + degp_ref[1] + 1.0
    dis = jnp.where(deg > 0, lax.rsqrt(deg), 0.0)
    out_ref[...] = dis[:, None]


def _dis(degp):
    return pl.pallas_call(
        _dis_body,
        out_shape=jax.ShapeDtypeStruct((N_PAD, 1), jnp.float32),
    )(degp)


def _combine_body(p_ref, prev_ref, dis_ref, out_ref):
    d = dis_ref[...]
    out_ref[...] = p_ref[0] + p_ref[1] + (d * d) * prev_ref[...]


def _combine(partials, prev, dis2d):
    return pl.pallas_call(
        _combine_body,
        out_shape=jax.ShapeDtypeStruct((N_PAD, D), jnp.float32),
    )(partials, prev, dis2d)


def _final_body(p_ref, prev_ref, dis_ref, w_ref, b_ref, out_ref):
    d = dis_ref[...]
    h = p_ref[0] + p_ref[1] + (d * d) * prev_ref[...]
    z = lax.dot_general(
        h, w_ref[...],
        dimension_numbers=(((1,), (1,)), ((), ())),
        preferred_element_type=jnp.float32,
        precision=lax.Precision.HIGHEST,
    ) + b_ref[...]
    z = jnp.maximum(z, 0.0)
    m = jnp.max(z, axis=1, keepdims=True)
    zc = z - m
    lse = jnp.log(jnp.sum(jnp.exp(zc), axis=1, keepdims=True))
    out_ref[...] = (zc - lse)[:N]


def _final(partials, prev, dis2d, w, b2):
    return pl.pallas_call(
        _final_body,
        out_shape=jax.ShapeDtypeStruct((N, D), jnp.float32),
    )(partials, prev, dis2d, w, b2)


_MESH = plsc.VectorSubcoreMesh(
    core_axis_name="c", subcore_axis_name="s", num_cores=NC, num_subcores=NS)

_SC_PARAMS = pltpu.CompilerParams()
if "needs_layout_passes" in pltpu.CompilerParams.__dataclass_fields__:
    _SC_PARAMS = dataclasses.replace(_SC_PARAMS, needs_layout_passes=False)


def _worker_id():
    return lax.axis_index("c") * NS + lax.axis_index("s")


def _deg_sc(dst3, ew3):
    ch_w = dst3.shape[1]

    @functools.partial(
        pl.kernel,
        out_type=jax.ShapeDtypeStruct((NC, N_PAD), jnp.float32),
        mesh=_MESH,
        compiler_params=_SC_PARAMS,
        scratch_types=[
            pltpu.VMEM_SHARED((N_PAD,), jnp.float32),
            pltpu.VMEM((ch_w, CHUNK), jnp.int32),
            pltpu.VMEM((ch_w, CHUNK), jnp.float32),
            pltpu.VMEM((ROWS_PER_TILE,), jnp.float32),
        ],
    )
    def body(dst_hbm, ew_hbm, out_hbm, acc, dstb, ewb, zb):
        c = lax.axis_index("c")
        s = lax.axis_index("s")
        w = _worker_id()
        my_rows = pl.ds(s * ROWS_PER_TILE, ROWS_PER_TILE)

        @pl.loop(0, ROWS_PER_TILE, step=LANES)
        def _(i):
            zb[pl.ds(i, LANES)] = jnp.zeros((LANES,), jnp.float32)

        pltpu.sync_copy(zb, acc.at[my_rows])
        plsc.subcore_barrier()

        pltpu.sync_copy(dst_hbm.at[w], dstb)
        pltpu.sync_copy(ew_hbm.at[w], ewb)

        @pl.loop(0, ch_w)
        def _(j):
            pltpu.sync_copy(ewb.at[j], acc.at[dstb.at[j]], add=True)

        plsc.subcore_barrier()
        pltpu.sync_copy(acc.at[my_rows], zb)
        pltpu.sync_copy(zb, out_hbm.at[c, my_rows])

    return body(dst3, ew3)


def _norm_sc(src3, dst3, ew3, dis1):
    ch_w = src3.shape[1]
    e_w = ch_w * CHUNK

    @functools.partial(
        pl.kernel,
        out_type=jax.ShapeDtypeStruct((NW, e_w), jnp.float32),
        mesh=_MESH,
        compiler_params=_SC_PARAMS,
        scratch_types=[
            pltpu.VMEM((N_PAD,), jnp.float32),
            pltpu.VMEM((ch_w, CHUNK), jnp.int32),
            pltpu.VMEM((ch_w, CHUNK), jnp.int32),
            pltpu.VMEM((ch_w, CHUNK), jnp.float32),
            pltpu.VMEM((e_w,), jnp.float32),
        ],
    )
    def body(src_hbm, dst_hbm, ew_hbm, dis_hbm, out_hbm,
             disb, srcb, dstb, ewb, nrmb):
        w = _worker_id()
        pltpu.sync_copy(dis_hbm, disb)
        pltpu.sync_copy(src_hbm.at[w], srcb)
        pltpu.sync_copy(dst_hbm.at[w], dstb)
        pltpu.sync_copy(ew_hbm.at[w], ewb)

        @pl.loop(0, ch_w)
        def _(j):
            @pl.loop(0, CHUNK // LANES)
            def _(g):
                sl = pl.ds(g * LANES, LANES)
                si = srcb[j, sl]
                di = dstb[j, sl]
                e16 = ewb[j, sl]
                nv = (plsc.load_gather(disb, [si]) * e16
                      * plsc.load_gather(disb, [di]))
                nrmb[pl.ds(j * CHUNK + g * LANES, LANES)] = nv

        pltpu.sync_copy(nrmb, out_hbm.at[w])

    return body(src3, dst3, ew3, dis1)


def _hop_sc(table, src3, dst3, norm2):
    ch_w = src3.shape[1]

    @functools.partial(
        pl.kernel,
        out_type=jax.ShapeDtypeStruct((NC, N_PAD, D), jnp.float32),
        mesh=_MESH,
        compiler_params=_SC_PARAMS,
        scratch_types=[
            pltpu.VMEM_SHARED((N_PAD, D), jnp.float32),
            pltpu.VMEM((ch_w, CHUNK), jnp.int32),
            pltpu.VMEM((ch_w, CHUNK), jnp.int32),
            pltpu.VMEM((ch_w * CHUNK,), jnp.float32),
            pltpu.VMEM((CHUNK, D), jnp.float32),
        ],
    )
    def body(table_hbm, src_hbm, dst_hbm, norm_hbm, out_hbm,
             acc, srcb, dstb, nrmb, gbuf):
        c = lax.axis_index("c")
        s = lax.axis_index("s")
        w = _worker_id()

        @pl.loop(0, CHUNK)
        def _(i):
            for k in range(D // LANES):
                gbuf[i, pl.ds(k * LANES, LANES)] = jnp.zeros(
                    (LANES,), jnp.float32)

        @pl.loop(0, ROWS_PER_TILE, step=CHUNK)
        def _(r):
            pltpu.sync_copy(gbuf, acc.at[pl.ds(s * ROWS_PER_TILE + r, CHUNK)])

        plsc.subcore_barrier()

        pltpu.sync_copy(src_hbm.at[w], srcb)
        pltpu.sync_copy(dst_hbm.at[w], dstb)
        pltpu.sync_copy(norm_hbm.at[w], nrmb)

        @pl.loop(0, ch_w)
        def _(j):
            pltpu.sync_copy(table_hbm.at[srcb.at[j]], gbuf)

            @pl.loop(0, CHUNK)
            def _(i):
                bc = plsc.load_gather(
                    nrmb, [jnp.full((LANES,), j * CHUNK + i, jnp.int32)])
                for k in range(D // LANES):
                    sl = pl.ds(k * LANES, LANES)
                    gbuf[i, sl] = gbuf[i, sl] * bc

            pltpu.sync_copy(gbuf, acc.at[dstb.at[j]], add=True)

        plsc.subcore_barrier()

        @pl.loop(0, ROWS_PER_TILE, step=CHUNK)
        def _(r):
            rows = pl.ds(s * ROWS_PER_TILE + r, CHUNK)
            pltpu.sync_copy(acc.at[rows], gbuf)
            pltpu.sync_copy(gbuf, out_hbm.at[c, rows])

    return body(table, src3, dst3, norm2)


def kernel(x, edge_index, edge_attr, W, b):
    e = edge_index.shape[1]
    ch_w = -(-e // (NW * CHUNK))
    e_pad = NW * CHUNK * ch_w
    pad = e_pad - e

    src = jnp.pad(edge_index[0], (0, pad)).reshape(NW, ch_w, CHUNK)
    dst = jnp.pad(edge_index[1], (0, pad)).reshape(NW, ch_w, CHUNK)
    ew = jnp.pad(_edge_weight(edge_attr), (0, pad)).reshape(NW, ch_w, CHUNK)

    degp = _deg_sc(dst, ew)
    dis2d = _dis(degp)
    norm2 = _norm_sc(src, dst, ew, dis2d.reshape(N_PAD))

    xp = jnp.pad(x, ((0, N_PAD - N), (0, 0)))
    p1 = _hop_sc(xp, src, dst, norm2)
    h1 = _combine(p1, xp, dis2d)
    p2 = _hop_sc(h1, src, dst, norm2)
    return _final(p2, h1, dis2d, W, b.reshape(1, D))
